# baseline (speedup 1.0000x reference)
import functools

import jax
import jax.numpy as jnp
from jax import lax
from jax.experimental import pallas as pl
from jax.experimental.pallas import tpu as pltpu
from jax.experimental.pallas import tpu_sc as plsc

_BATCH = 4
_SEQ = 8192
_D = 1024
_NUM_CORES = 2
_NUM_SUBCORES = 16
_NW = _NUM_CORES * _NUM_SUBCORES
_ROWS_PER_W = _SEQ // _NW
_CHUNK = 64
_NCHUNK = _ROWS_PER_W // _CHUNK


@functools.partial(
    pl.kernel,
    mesh=plsc.VectorSubcoreMesh(core_axis_name="c", subcore_axis_name="s"),
    out_type=jax.ShapeDtypeStruct((_BATCH * _SEQ, _D), jnp.float32),
    scratch_types=[pltpu.VMEM((_CHUNK, _D), jnp.float32)],
)
def _sc_copy(emb_hbm, out_hbm, buf):
    wid = lax.axis_index("s") * _NUM_CORES + lax.axis_index("c")
    base = wid * _ROWS_PER_W
    for c in range(_NCHUNK):
        off = base + c * _CHUNK
        pltpu.sync_copy(emb_hbm.at[pl.ds(off, _CHUNK)], buf)
        for b in range(_BATCH):
            pltpu.sync_copy(buf, out_hbm.at[pl.ds(b * _SEQ + off, _CHUNK)])


def kernel(x, emb):
    batch, seq_len, d_model = x.shape
    out = _sc_copy(emb[:seq_len])
    return out.reshape(batch, seq_len, d_model)
